# baseline (speedup 1.0000x reference)
import functools

import jax
import jax.numpy as jnp
from jax import lax
from jax.experimental import pallas as pl
from jax.experimental.pallas import tpu as pltpu
from jax.experimental.pallas import tpu_sc as plsc

B = 1024
CTX = 20
D = 64
V = 100000

NC = 2
NS = 16
NW = NC * NS
BPW = B // NW
IDX_PER_W = BPW * CTX

_sc_mesh = plsc.VectorSubcoreMesh(core_axis_name="c", subcore_axis_name="s")


@functools.partial(
    pl.kernel,
    mesh=_sc_mesh,
    out_type=jax.ShapeDtypeStruct((B * CTX, 128), jnp.float32),
    scratch_types=[
        pltpu.VMEM((IDX_PER_W,), jnp.int32),
        pltpu.VMEM((IDX_PER_W, 128), jnp.float32),
        pltpu.SemaphoreType.DMA,
    ],
)
def _gather(idx_hbm, table_hbm, out_hbm, idx_v, rows_v, sem):
    wid = lax.axis_index("s") * NC + lax.axis_index("c")
    base = wid * IDX_PER_W
    pltpu.sync_copy(idx_hbm.at[pl.ds(base, IDX_PER_W)], idx_v)
    pltpu.async_copy(table_hbm.at[idx_v], rows_v, sem).wait()
    pltpu.sync_copy(rows_v, out_hbm.at[pl.ds(base, IDX_PER_W)])


BV = 3072


def _proj_body(wt_ref, rows_ref, par_ref, brow_ref, out_ref, cb_ref):
    @pl.when(pl.program_id(0) == 0)
    def _():
        x = rows_ref[...]
        p = par_ref[...]
        acc = jnp.zeros((B, D), jnp.float32)
        for j in range(CTX):
            lo = x[:, j, :D]
            hi = x[:, j, D:]
            acc = acc + (lo + p[:, j:j + 1] * (hi - lo))
        cb_ref[...] = acc

    acc = lax.dot_general(
        wt_ref[...], cb_ref[...],
        dimension_numbers=(((0,), (1,)), ((), ())),
        preferred_element_type=jnp.float32,
    )
    bias_t = lax.dot_general(
        brow_ref[...], jnp.ones((1, B), jnp.float32),
        dimension_numbers=(((0,), (0,)), ((), ())),
        preferred_element_type=jnp.float32,
    )
    out_ref[...] = acc + bias_t


def _projection_t(wt, rows3, par, brow):
    nv = pl.cdiv(V, BV)
    return pl.pallas_call(
        _proj_body,
        grid=(nv,),
        in_specs=[
            pl.BlockSpec((D, BV), lambda i: (0, i)),
            pl.BlockSpec((B, CTX, 128), lambda i: (0, 0, 0)),
            pl.BlockSpec((B, CTX), lambda i: (0, 0)),
            pl.BlockSpec((1, BV), lambda i: (0, i)),
        ],
        out_specs=pl.BlockSpec((BV, B), lambda i: (i, 0)),
        out_shape=jax.ShapeDtypeStruct((V, B), jnp.float32),
        scratch_shapes=[pltpu.VMEM((B, D), jnp.float32)],
    )(wt, rows3, par, brow)


def kernel(inputs, emb_table, W, b):
    idx = inputs.astype(jnp.int32).reshape(-1)
    table2 = emb_table.reshape(V // 2, 2 * D)
    rows = _gather(idx >> 1, table2)
    rows3 = rows.reshape(B, CTX, 128)
    par = (inputs & 1).astype(jnp.float32)
    out_t = _projection_t(W.T, rows3, par, b.reshape(1, V))
    return out_t.T


# speedup vs baseline: 1.1090x; 1.1090x over previous
import functools

import jax
import jax.numpy as jnp
from jax import lax
from jax.experimental import pallas as pl
from jax.experimental.pallas import tpu as pltpu
from jax.experimental.pallas import tpu_sc as plsc

B = 1024
CTX = 20
D = 64
V = 100000

NC = 2
NS = 16
NW = NC * NS
BPW = B // NW
IDX_PER_W = BPW * CTX

_sc_mesh = plsc.VectorSubcoreMesh(core_axis_name="c", subcore_axis_name="s")


@functools.partial(
    pl.kernel,
    mesh=_sc_mesh,
    out_type=jax.ShapeDtypeStruct((B, D), jnp.float32),
    scratch_types=[
        pltpu.VMEM((IDX_PER_W,), jnp.int32),
        pltpu.VMEM((IDX_PER_W, 128), jnp.float32),
        pltpu.VMEM((BPW, D), jnp.float32),
        pltpu.SemaphoreType.DMA,
    ],
)
def _gather_sum(idx_hbm, table_hbm, out_hbm, idx_v, rows_v, acc_v, sem):
    wid = lax.axis_index("s") * NC + lax.axis_index("c")
    base = wid * BPW
    pltpu.sync_copy(idx_hbm.at[pl.ds(base * CTX, IDX_PER_W)], idx_v)
    pltpu.async_copy(table_hbm.at[idx_v], rows_v, sem).wait()

    def body(bi, carry):
        rbase = bi * CTX
        for k in range(D // 16):
            acc = rows_v[rbase, pl.ds(k * 16, 16)]
            for j in range(1, CTX):
                acc = acc + rows_v[rbase + j, pl.ds(k * 16, 16)]
            acc_v[bi, pl.ds(k * 16, 16)] = acc
        return carry

    lax.fori_loop(0, BPW, body, 0)
    pltpu.sync_copy(acc_v, out_hbm.at[pl.ds(base, BPW)])


BV = 5120


def _proj_body(wt_ref, emb_ref, brow_ref, out_ref):
    acc = lax.dot_general(
        wt_ref[...], emb_ref[...],
        dimension_numbers=(((0,), (1,)), ((), ())),
        preferred_element_type=jnp.float32,
    )
    bias_t = lax.dot_general(
        brow_ref[...], jnp.ones((1, B), jnp.float32),
        dimension_numbers=(((0,), (0,)), ((), ())),
        preferred_element_type=jnp.float32,
    )
    out_ref[...] = acc + bias_t


def _projection_t(wt, cbow, brow):
    nv = pl.cdiv(V, BV)
    return pl.pallas_call(
        _proj_body,
        grid=(nv,),
        in_specs=[
            pl.BlockSpec((D, BV), lambda i: (0, i)),
            pl.BlockSpec((B, D), lambda i: (0, 0)),
            pl.BlockSpec((1, BV), lambda i: (0, i)),
        ],
        out_specs=pl.BlockSpec((BV, B), lambda i: (i, 0)),
        out_shape=jax.ShapeDtypeStruct((V, B), jnp.float32),
    )(wt, cbow, brow)


def kernel(inputs, emb_table, W, b):
    idx = inputs.astype(jnp.int32).reshape(-1)
    table_p = jnp.concatenate([emb_table, emb_table], axis=1)
    cbow = _gather_sum(idx, table_p)
    out_t = _projection_t(W.T, cbow, b.reshape(1, V))
    return out_t.T


# speedup vs baseline: 1.1968x; 1.0792x over previous
import functools

import jax
import jax.numpy as jnp
from jax import lax
from jax.experimental import pallas as pl
from jax.experimental.pallas import tpu as pltpu
from jax.experimental.pallas import tpu_sc as plsc

B = 1024
CTX = 20
D = 64
V = 100000

NC = 2
NS = 16
NW = NC * NS
BPW = B // NW
IDX_PER_W = BPW * CTX

_sc_mesh = plsc.VectorSubcoreMesh(core_axis_name="c", subcore_axis_name="s")


@functools.partial(
    pl.kernel,
    mesh=_sc_mesh,
    out_type=jax.ShapeDtypeStruct((B, D), jnp.float32),
    scratch_types=[
        pltpu.VMEM((IDX_PER_W,), jnp.int32),
        pltpu.VMEM((IDX_PER_W, 128), jnp.float32),
        pltpu.VMEM((BPW, D), jnp.float32),
        pltpu.SemaphoreType.DMA,
        pltpu.SemaphoreType.DMA,
    ],
)
def _gather_sum(idx_hbm, table_hbm, out_hbm, idx_v, rows_v, acc_v, s0, s1):
    wid = lax.axis_index("s") * NC + lax.axis_index("c")
    base = wid * BPW
    half = IDX_PER_W // 2
    pltpu.sync_copy(idx_hbm.at[pl.ds(base * CTX, IDX_PER_W)], idx_v)
    h0 = pltpu.async_copy(
        table_hbm.at[idx_v.at[pl.ds(0, half)]], rows_v.at[pl.ds(0, half)], s0)
    h1 = pltpu.async_copy(
        table_hbm.at[idx_v.at[pl.ds(half, half)]],
        rows_v.at[pl.ds(half, half)], s1)

    def body(bi, carry):
        rbase = bi * CTX
        for k in range(D // 16):
            acc = rows_v[rbase, pl.ds(k * 16, 16)]
            for j in range(1, CTX):
                acc = acc + rows_v[rbase + j, pl.ds(k * 16, 16)]
            acc_v[bi, pl.ds(k * 16, 16)] = acc
        return carry

    h0.wait()
    lax.fori_loop(0, BPW // 2, body, 0)
    h1.wait()
    lax.fori_loop(BPW // 2, BPW, body, 0)
    pltpu.sync_copy(acc_v, out_hbm.at[pl.ds(base, BPW)])


BV = 5120


def _proj_body(wt_ref, emb_ref, brow_ref, out_ref):
    acc = lax.dot_general(
        wt_ref[...], emb_ref[...],
        dimension_numbers=(((0,), (1,)), ((), ())),
        preferred_element_type=jnp.float32,
    )
    bias_t = lax.dot_general(
        brow_ref[...], jnp.ones((1, B), jnp.float32),
        dimension_numbers=(((0,), (0,)), ((), ())),
        preferred_element_type=jnp.float32,
    )
    out_ref[...] = acc + bias_t


def _projection_t(wt, cbow, brow):
    nv = pl.cdiv(V, BV)
    return pl.pallas_call(
        _proj_body,
        grid=(nv,),
        in_specs=[
            pl.BlockSpec((D, BV), lambda i: (0, i)),
            pl.BlockSpec((B, D), lambda i: (0, 0)),
            pl.BlockSpec((1, BV), lambda i: (0, i)),
        ],
        out_specs=pl.BlockSpec((BV, B), lambda i: (i, 0)),
        out_shape=jax.ShapeDtypeStruct((V, B), jnp.float32),
    )(wt, cbow, brow)


def kernel(inputs, emb_table, W, b):
    idx = inputs.astype(jnp.int32).reshape(-1)
    table_p = jnp.pad(emb_table, ((0, 0), (0, 128 - D)))
    cbow = _gather_sum(idx, table_p)
    out_t = _projection_t(W.T, cbow, b.reshape(1, V))
    return out_t.T
